# baseline (speedup 1.0000x reference)
import functools

import jax
import jax.numpy as jnp
from jax import lax
from jax.experimental import pallas as pl
from jax.experimental.pallas import tpu as pltpu
from jax.experimental.pallas import tpu_sc as plsc

N = 50000
M = 10000
K = 16

NC = 2
NS = 16
NW = NC * NS
L = 16

NPW = 1568
NP = NPW * NW
EPW = NPW * K
CP = 8
EC = CP * K
NCH = NPW // CP
NBUF = 4

BR = 512
MPAD = 10240

_SC_PARAMS = pltpu.CompilerParams(needs_layout_passes=False,
                                  use_tc_tiling_on_sc=False)


def _mesh():
    return plsc.VectorSubcoreMesh(core_axis_name="c", subcore_axis_name="s")


def _wid():
    return lax.axis_index("s") * NC + lax.axis_index("c")


def _coord_body(c, ob, sb, tab, qb):
    def body(p, carry):
        ev = ob[pl.ds(p * K, K)]
        raw = ev if c == 0 else jax.lax.shift_right_logical(ev, 2)
        xs = plsc.load_gather(tab, [raw])
        qsplat = plsc.load_gather(qb, [jnp.zeros((K,), jnp.int32) + p])
        rel = xs - qsplat
        r2 = rel * rel
        pos = (rel > 0).astype(jnp.int32)
        if c == 0:
            sb[pl.ds(p * K, K)] = r2
            ob[pl.ds(p * K, K)] = ev * 4 + pos * 2
        elif c == 1:
            sb[pl.ds(p * K, K)] = sb[pl.ds(p * K, K)] + r2
            ob[pl.ds(p * K, K)] = ev + pos
        else:
            sb[pl.ds(p * K, K)] = jnp.exp(-(sb[pl.ds(p * K, K)] + r2))
            ob[pl.ds(p * K, K)] = (jax.lax.shift_right_logical(ev, 2)
                                   + (ev & 3) * NP)
        return carry
    return body


def _edge_phase(tabs, tlen, qsrcs, idx_in, ebase, pbase, ob, sb, tab, qb):
    pltpu.sync_copy(idx_in.at[pl.ds(ebase, EPW)], ob)
    for c in range(3):
        pltpu.sync_copy(tabs[c], tab.at[pl.ds(0, tlen)])
        pltpu.sync_copy(qsrcs[c].at[pl.ds(pbase, NPW)],
                        qb.at[pl.ds(0, NPW)])
        lax.fori_loop(0, NPW, _coord_body(c, ob, sb, tab, qb), 0, unroll=4)


BOXSTRIDE = 10240


def _box_phase(btabs, qsrcs, idx_in, ebase, pbase, ob, sb, tab, qb):
    pltpu.sync_copy(idx_in.at[pl.ds(ebase, EPW)], ob)
    for t in range(3):
        pltpu.sync_copy(btabs[t], tab.at[pl.ds(t * BOXSTRIDE, M)])
        pltpu.sync_copy(qsrcs[t].at[pl.ds(pbase, NPW)],
                        qb.at[pl.ds(t * NPW, NPW)])

    def body(p, carry):
        ev = ob[pl.ds(p * K, K)]
        iq = jnp.zeros((K,), jnp.int32) + p
        s = None
        bb = None
        for t in range(3):
            xs = plsc.load_gather(tab.at[pl.ds(t * BOXSTRIDE, M)], [ev])
            q = plsc.load_gather(qb.at[pl.ds(t * NPW, NPW)], [iq])
            rel = xs - q
            r2 = rel * rel
            s = r2 if t == 0 else s + r2
            if t == 0:
                bb = (rel > 0).astype(jnp.int32) * 2
            elif t == 1:
                bb = bb + (rel > 0).astype(jnp.int32)
        sb[pl.ds(p * K, K)] = jnp.exp(-s)
        ob[pl.ds(p * K, K)] = ev + bb * MPAD
        return carry

    lax.fori_loop(0, NPW, body, 0, unroll=4)


def _gather_sum(ytab, out, offb, wb, rows, outb, gsems, osems, pbase, C,
                nbuf, out_col=0):
    nsub = C // L

    def odst(ch):
        return out.at[pl.ds(pbase + ch * CP, CP), pl.ds(out_col, C)]

    def issue(ch, j):
        pltpu.async_copy(
            ytab.at[offb.at[pl.ds(ch * EC, EC)]], rows.at[j], gsems[j])

    for j in range(nbuf):
        issue(j, j)

    def outer(g, carry):
        for j in range(nbuf):
            ch = g * nbuf + j
            pltpu.make_async_copy(
                ytab.at[offb.at[pl.ds(ch * EC, EC)]], rows.at[j],
                gsems[j]).wait()

            @pl.when(ch >= nbuf)
            def _():
                pltpu.make_async_copy(
                    outb.at[j], odst(ch - nbuf), osems[j]).wait()

            def acc_body(p, inner):
                e0 = ch * EC + p * K
                wv = wb[pl.ds(e0, K)]
                for cb in range(nsub):
                    ts = [wv[kk] * rows[j, p * K + kk, pl.ds(cb * L, L)]
                          for kk in range(K)]
                    while len(ts) > 1:
                        ts = [ts[i] + ts[i + 1] for i in range(0, len(ts), 2)]
                    outb[j, p, pl.ds(cb * L, L)] = ts[0]
                return inner

            lax.fori_loop(0, CP, acc_body, 0)

            @pl.when(ch + nbuf < NCH)
            def _():
                issue(ch + nbuf, j)

            pltpu.async_copy(outb.at[j], odst(ch), osems[j])
        return carry

    lax.fori_loop(0, NCH // nbuf, outer, 0)
    for ch in range(NCH - nbuf, NCH):
        pltpu.make_async_copy(
            outb.at[ch % nbuf], odst(ch), osems[ch % nbuf]).wait()


@functools.partial(
    pl.kernel,
    out_type=[
        jax.ShapeDtypeStruct((NP * K,), jnp.int32),
        jax.ShapeDtypeStruct((NP * K,), jnp.float32),
        jax.ShapeDtypeStruct((NP, 128), jnp.float32),
    ],
    mesh=_mesh(),
    scratch_types=[
        pltpu.VMEM((NP,), jnp.float32),
        pltpu.VMEM((3 * NPW,), jnp.float32),
        pltpu.VMEM((EPW,), jnp.int32),
        pltpu.VMEM((EPW,), jnp.float32),
        pltpu.VMEM((4, EC, 32), jnp.float32),
        pltpu.VMEM((4, CP, 32), jnp.float32),
    ] + [pltpu.SemaphoreType.DMA] * 8,
    compiler_params=_SC_PARAMS,
)
def _stage1(dyx, dyy, dyz, bxx, bxy, bxz, dyi, bxi, y1b, y1d,
            dyo, dyw, ccb,
            tab, qb, ob, sb, rows, outb,
            g0, g1, g2, g3, o0, o1, o2, o3):
    wid = _wid()
    ebase = wid * EPW
    pbase = wid * NPW
    gsems = (g0, g1, g2, g3)
    osems = (o0, o1, o2, o3)
    qsrcs = (dyx, dyy, dyz)

    _box_phase((bxx, bxy, bxz), qsrcs, bxi, ebase, pbase, ob, sb, tab, qb)
    _gather_sum(y1b, ccb, ob, sb, rows, outb, gsems, osems, pbase, 32,
                nbuf=4, out_col=0)

    _edge_phase(qsrcs, NP, qsrcs, dyi, ebase, pbase, ob, sb, tab, qb)
    pltpu.sync_copy(ob, dyo.at[pl.ds(ebase, EPW)])
    pltpu.sync_copy(sb, dyw.at[pl.ds(ebase, EPW)])
    _gather_sum(y1d, ccb, ob, sb, rows, outb, gsems, osems, pbase, 32,
                nbuf=4, out_col=32)


GNBUF = 7


def _make_gather(C):
    @functools.partial(
        pl.kernel,
        out_type=jax.ShapeDtypeStruct((NP, 128), jnp.float32),
        mesh=_mesh(),
        scratch_types=[
            pltpu.VMEM((EPW,), jnp.int32),
            pltpu.VMEM((EPW,), jnp.float32),
            pltpu.VMEM((GNBUF, EC, C), jnp.float32),
            pltpu.VMEM((GNBUF, CP, C), jnp.float32),
        ] + [pltpu.SemaphoreType.DMA] * (2 * GNBUF),
        compiler_params=_SC_PARAMS,
    )
    def k(ytab, off, w, out, offb, wb, rows, outb,
          g0, g1, g2, g3, g4, g5, g6, o0, o1, o2, o3, o4, o5, o6):
        wid = _wid()
        ebase = wid * EPW
        pltpu.sync_copy(off.at[pl.ds(ebase, EPW)], offb)
        pltpu.sync_copy(w.at[pl.ds(ebase, EPW)], wb)
        _gather_sum(ytab, out, offb, wb, rows, outb,
                    (g0, g1, g2, g3, g4, g5, g6),
                    (o0, o1, o2, o3, o4, o5, o6), wid * NPW, C, nbuf=GNBUF)

    return k


_gather64 = _make_gather(64)
_gather32 = _make_gather(32)


def _dot(a, b):
    return jax.lax.dot_general(
        a, b, (((1,), (0,)), ((), ())),
        preferred_element_type=jnp.float32)


def _tab_dot(x, w, o):
    for b in range(4):
        o[b, :, :] = _dot(x, w[b, :, :])


def _mm_body(x, w, o):
    _tab_dot(x[...], w, o)


def _tc_tab_matmul(x, w):
    R, Cin = x.shape
    Cout = w.shape[2]
    return pl.pallas_call(
        _mm_body,
        grid=(R // BR,),
        in_specs=[pl.BlockSpec((BR, Cin), lambda i: (i, 0)),
                  pl.BlockSpec((4, Cin, Cout), lambda i: (0, 0, 0))],
        out_specs=pl.BlockSpec((4, BR, Cout), lambda i: (0, i, 0)),
        out_shape=jax.ShapeDtypeStruct((4, R, Cout), jnp.float32),
    )(x, w)


def _l1_body(ccb, ft, f1w, f1b, w2, x1o, y2o):
    self1 = _dot(ft[...], f1w[...]) + f1b[...]
    x1 = jnp.maximum(
        jnp.concatenate([ccb[:, :64], self1], axis=1), 0.0)
    x1o[...] = x1
    _tab_dot(x1, w2, y2o)


def _l2_body(cc2, x1, f2w, f2b, w3, x2o, y3o):
    x2 = (jnp.maximum(cc2[:, :64], 0.0) + _dot(x1[...], f2w[...])
          + f2b[...])
    x2o[...] = x2
    _tab_dot(x2, w3, y3o)


def _l3_body(cc3, x2, f3w, f3b, w4, x3o, y4o):
    x3 = _dot(x2[...], f3w[...]) + f3b[...] + cc3[:, :64]
    x3o[...] = x3
    _tab_dot(x3, w4, y4o)


def _l4_body(cc4, x3, f4w, f4b, xo):
    xo[...] = _dot(x3[...], f4w[...]) + f4b[...] + cc4[:, :16]


def _row_spec(c):
    return pl.BlockSpec((BR, c), lambda i: (i, 0))


def _full_spec(shape):
    n = len(shape)
    return pl.BlockSpec(shape, lambda i: (0,) * n)


def _tab_spec(c):
    return pl.BlockSpec((4, BR, c), lambda i: (0, i, 0))


def _tc_fused(body, ins, outs):
    return pl.pallas_call(
        body,
        grid=(NP // BR,),
        in_specs=[s for _, s in ins],
        out_specs=[s for _, s in outs],
        out_shape=[jax.ShapeDtypeStruct(sh, jnp.float32) for sh, _ in outs],
    )(*[a for a, _ in ins])


def kernel(dy_positions, dy_feats, box_positions, box_feats, dy_indxs,
           box_indxs, W_cc1, W_cc2, W_cc3, W_cc4,
           fc1_w, fc1_b, fc2_w, fc2_b, fc3_w, fc3_b, fc4_w, fc4_b):
    dyp = jnp.pad(dy_positions, ((0, NP - N), (0, 0)))
    dyf = jnp.pad(dy_feats, ((0, NP - N), (0, 6)))
    bxf = jnp.pad(box_feats, ((0, MPAD - M), (0, 6)))
    dyi = jnp.pad(dy_indxs, ((0, NP - N), (0, 0))).reshape(-1)
    bxi = jnp.pad(box_indxs, ((0, NP - N), (0, 0))).reshape(-1)
    dyx, dyy, dyz = dyp[:, 0], dyp[:, 1], dyp[:, 2]
    bxx, bxy, bxz = (box_positions[:, 0], box_positions[:, 1],
                     box_positions[:, 2])

    w1p = jnp.pad(W_cc1, ((0, 0), (0, 6), (0, 0)))
    w4p = jnp.pad(W_cc4, ((0, 0), (0, 0), (0, 29)))
    f1w = jnp.pad(fc1_w, ((0, 6), (0, 0)))
    f4w = jnp.pad(fc4_w, ((0, 0), (0, 13)))
    f4b = jnp.pad(fc4_b, (0, 13))

    y1d = _tc_tab_matmul(dyf, w1p)
    y1b = _tc_tab_matmul(bxf, w1p)
    dyo, dyw, ccb = _stage1(dyx, dyy, dyz, bxx, bxy, bxz, dyi, bxi,
                            y1b.reshape(4 * MPAD, 32),
                            y1d.reshape(4 * NP, 32))
    x1, y2 = _tc_fused(
        _l1_body,
        [(ccb, _row_spec(128)), (dyf, _row_spec(8)),
         (f1w, _full_spec((8, 32))),
         (fc1_b.reshape(1, 32), _full_spec((1, 32))),
         (W_cc2, _full_spec((4, 96, 64)))],
        [((NP, 96), _row_spec(96)), ((4, NP, 64), _tab_spec(64))])

    cc2 = _gather64(y2.reshape(4 * NP, 64), dyo, dyw)
    x2, y3 = _tc_fused(
        _l2_body,
        [(cc2, _row_spec(128)), (x1, _row_spec(96)),
         (fc2_w, _full_spec((96, 64))),
         (fc2_b.reshape(1, 64), _full_spec((1, 64))),
         (W_cc3, _full_spec((4, 64, 64)))],
        [((NP, 64), _row_spec(64)), ((4, NP, 64), _tab_spec(64))])

    cc3 = _gather64(y3.reshape(4 * NP, 64), dyo, dyw)
    x3, y4 = _tc_fused(
        _l3_body,
        [(cc3, _row_spec(128)), (x2, _row_spec(64)),
         (fc3_w, _full_spec((64, 64))),
         (fc3_b.reshape(1, 64), _full_spec((1, 64))),
         (w4p, _full_spec((4, 64, 32)))],
        [((NP, 64), _row_spec(64)), ((4, NP, 32), _tab_spec(32))])

    cc4 = _gather32(y4.reshape(4 * NP, 32), dyo, dyw)
    (x4,) = _tc_fused(
        _l4_body,
        [(cc4, _row_spec(128)), (x3, _row_spec(64)),
         (f4w, _full_spec((64, 16))),
         (f4b.reshape(1, 16), _full_spec((1, 16)))],
        [((NP, 16), _row_spec(16))])

    return x4[:N, :3]


# speedup vs baseline: 1.1295x; 1.1295x over previous
import functools

import jax
import jax.numpy as jnp
from jax import lax
from jax.experimental import pallas as pl
from jax.experimental.pallas import tpu as pltpu
from jax.experimental.pallas import tpu_sc as plsc

N = 50000
M = 10000
K = 16

NC = 2
NS = 16
NW = NC * NS
L = 16

NPW = 1568
NP = NPW * NW
EPW = NPW * K
CP = 8
EC = CP * K
NCH = NPW // CP
NBUF = 4

BR = 512
MPAD = 10240

_SC_PARAMS = pltpu.CompilerParams(needs_layout_passes=False,
                                  use_tc_tiling_on_sc=False)


def _mesh():
    return plsc.VectorSubcoreMesh(core_axis_name="c", subcore_axis_name="s")


def _wid():
    return lax.axis_index("s") * NC + lax.axis_index("c")


def _coord_body(c, ob, sb, tab, qb):
    def body(p, carry):
        ev = ob[pl.ds(p * K, K)]
        raw = ev if c == 0 else jax.lax.shift_right_logical(ev, 2)
        xs = plsc.load_gather(tab, [raw])
        qsplat = plsc.load_gather(qb, [jnp.zeros((K,), jnp.int32) + p])
        rel = xs - qsplat
        r2 = rel * rel
        pos = (rel > 0).astype(jnp.int32)
        if c == 0:
            sb[pl.ds(p * K, K)] = r2
            ob[pl.ds(p * K, K)] = ev * 4 + pos * 2
        elif c == 1:
            sb[pl.ds(p * K, K)] = sb[pl.ds(p * K, K)] + r2
            ob[pl.ds(p * K, K)] = ev + pos
        else:
            sb[pl.ds(p * K, K)] = jnp.exp(-(sb[pl.ds(p * K, K)] + r2))
        return carry
    return body


def _edge_phase(tabs, tlen, qsrcs, idx_in, ebase, pbase, ob, sb, tab, qb):
    pltpu.sync_copy(idx_in.at[pl.ds(ebase, EPW)], ob)
    for c in range(3):
        pltpu.sync_copy(tabs[c], tab.at[pl.ds(0, tlen)])
        pltpu.sync_copy(qsrcs[c].at[pl.ds(pbase, NPW)],
                        qb.at[pl.ds(0, NPW)])
        lax.fori_loop(0, NPW, _coord_body(c, ob, sb, tab, qb), 0, unroll=4)


BOXSTRIDE = 10240


def _box_phase(btabs, qsrcs, idx_in, ebase, pbase, ob, sb, tab, qb):
    pltpu.sync_copy(idx_in.at[pl.ds(ebase, EPW)], ob)
    for t in range(3):
        pltpu.sync_copy(btabs[t], tab.at[pl.ds(t * BOXSTRIDE, M)])
        pltpu.sync_copy(qsrcs[t].at[pl.ds(pbase, NPW)],
                        qb.at[pl.ds(t * NPW, NPW)])

    def body(p, carry):
        ev = ob[pl.ds(p * K, K)]
        iq = jnp.zeros((K,), jnp.int32) + p
        s = None
        bb = None
        for t in range(3):
            xs = plsc.load_gather(tab.at[pl.ds(t * BOXSTRIDE, M)], [ev])
            q = plsc.load_gather(qb.at[pl.ds(t * NPW, NPW)], [iq])
            rel = xs - q
            r2 = rel * rel
            s = r2 if t == 0 else s + r2
            if t == 0:
                bb = (rel > 0).astype(jnp.int32) * 2
            elif t == 1:
                bb = bb + (rel > 0).astype(jnp.int32)
        sb[pl.ds(p * K, K)] = jnp.exp(-s)
        ob[pl.ds(p * K, K)] = ev * 4 + bb
        return carry

    lax.fori_loop(0, NPW, body, 0, unroll=4)


def _gather_sum(ytab, out, offb, wb, rows, outb, gsems, osems, pbase, C,
                nbuf, out_col=0):
    nsub = C // L

    def odst(ch):
        return out.at[pl.ds(pbase + ch * CP, CP), pl.ds(out_col, C)]

    def issue(ch, j):
        pltpu.async_copy(
            ytab.at[offb.at[pl.ds(ch * EC, EC)]], rows.at[j], gsems[j])

    for j in range(nbuf):
        issue(j, j)

    def outer(g, carry):
        for j in range(nbuf):
            ch = g * nbuf + j
            pltpu.make_async_copy(
                ytab.at[offb.at[pl.ds(ch * EC, EC)]], rows.at[j],
                gsems[j]).wait()

            @pl.when(ch >= nbuf)
            def _():
                pltpu.make_async_copy(
                    outb.at[j], odst(ch - nbuf), osems[j]).wait()

            def acc_body(p, inner):
                e0 = ch * EC + p * K
                wv = wb[pl.ds(e0, K)]
                for cb in range(nsub):
                    ts = [wv[kk] * rows[j, p * K + kk, pl.ds(cb * L, L)]
                          for kk in range(K)]
                    while len(ts) > 1:
                        ts = [ts[i] + ts[i + 1] for i in range(0, len(ts), 2)]
                    outb[j, p, pl.ds(cb * L, L)] = ts[0]
                return inner

            lax.fori_loop(0, CP, acc_body, 0)

            @pl.when(ch + nbuf < NCH)
            def _():
                issue(ch + nbuf, j)

            pltpu.async_copy(outb.at[j], odst(ch), osems[j])
        return carry

    lax.fori_loop(0, NCH // nbuf, outer, 0)
    for ch in range(NCH - nbuf, NCH):
        pltpu.make_async_copy(
            outb.at[ch % nbuf], odst(ch), osems[ch % nbuf]).wait()


@functools.partial(
    pl.kernel,
    out_type=[
        jax.ShapeDtypeStruct((NP * K,), jnp.int32),
        jax.ShapeDtypeStruct((NP * K,), jnp.float32),
        jax.ShapeDtypeStruct((NP, 128), jnp.float32),
    ],
    mesh=_mesh(),
    scratch_types=[
        pltpu.VMEM((NP,), jnp.float32),
        pltpu.VMEM((3 * NPW,), jnp.float32),
        pltpu.VMEM((EPW,), jnp.int32),
        pltpu.VMEM((EPW,), jnp.float32),
        pltpu.VMEM((4, EC, 32), jnp.float32),
        pltpu.VMEM((4, CP, 32), jnp.float32),
    ] + [pltpu.SemaphoreType.DMA] * 8,
    compiler_params=_SC_PARAMS,
)
def _stage1(dyx, dyy, dyz, bxx, bxy, bxz, dyi, bxi, y1b, y1d,
            dyo, dyw, ccb,
            tab, qb, ob, sb, rows, outb,
            g0, g1, g2, g3, o0, o1, o2, o3):
    wid = _wid()
    ebase = wid * EPW
    pbase = wid * NPW
    gsems = (g0, g1, g2, g3)
    osems = (o0, o1, o2, o3)
    qsrcs = (dyx, dyy, dyz)

    _box_phase((bxx, bxy, bxz), qsrcs, bxi, ebase, pbase, ob, sb, tab, qb)
    _gather_sum(y1b, ccb, ob, sb, rows, outb, gsems, osems, pbase, 32,
                nbuf=4, out_col=0)

    _edge_phase(qsrcs, NP, qsrcs, dyi, ebase, pbase, ob, sb, tab, qb)
    pltpu.sync_copy(ob, dyo.at[pl.ds(ebase, EPW)])
    pltpu.sync_copy(sb, dyw.at[pl.ds(ebase, EPW)])
    _gather_sum(y1d, ccb, ob, sb, rows, outb, gsems, osems, pbase, 32,
                nbuf=4, out_col=32)


GNBUF = 7


def _make_gather(C):
    @functools.partial(
        pl.kernel,
        out_type=jax.ShapeDtypeStruct((NP, 128), jnp.float32),
        mesh=_mesh(),
        scratch_types=[
            pltpu.VMEM((EPW,), jnp.int32),
            pltpu.VMEM((EPW,), jnp.float32),
            pltpu.VMEM((GNBUF, EC, C), jnp.float32),
            pltpu.VMEM((GNBUF, CP, C), jnp.float32),
        ] + [pltpu.SemaphoreType.DMA] * (2 * GNBUF),
        compiler_params=_SC_PARAMS,
    )
    def k(ytab, off, w, out, offb, wb, rows, outb,
          g0, g1, g2, g3, g4, g5, g6, o0, o1, o2, o3, o4, o5, o6):
        wid = _wid()
        ebase = wid * EPW
        pltpu.sync_copy(off.at[pl.ds(ebase, EPW)], offb)
        pltpu.sync_copy(w.at[pl.ds(ebase, EPW)], wb)
        _gather_sum(ytab, out, offb, wb, rows, outb,
                    (g0, g1, g2, g3, g4, g5, g6),
                    (o0, o1, o2, o3, o4, o5, o6), wid * NPW, C, nbuf=GNBUF)

    return k


_gather64 = _make_gather(64)
_gather32 = _make_gather(32)


def _dot(a, b):
    return jax.lax.dot_general(
        a, b, (((1,), (0,)), ((), ())),
        preferred_element_type=jnp.float32)


def _y1_body(ft, w1c, o):
    f = ft[...]
    w = w1c[...]
    o[...] = f[:, 0:1] * w[0:1, :] + f[:, 1:2] * w[1:2, :]


def _tc_y1(x, w1c):
    R = x.shape[0]
    return pl.pallas_call(
        _y1_body,
        grid=(R // BR,),
        in_specs=[pl.BlockSpec((BR, 8), lambda i: (i, 0)),
                  pl.BlockSpec((2, 128), lambda i: (0, 0))],
        out_specs=pl.BlockSpec((BR, 128), lambda i: (i, 0)),
        out_shape=jax.ShapeDtypeStruct((R, 128), jnp.float32),
    )(x, w1c)


def _l1_body(ccb, ft, f1w, f1b, w2c, x1o, y2o):
    self1 = _dot(ft[...], f1w[...]) + f1b[...]
    x1 = jnp.maximum(
        jnp.concatenate([ccb[:, :64], self1], axis=1), 0.0)
    x1o[...] = x1
    y2o[...] = _dot(x1, w2c[...])


def _l2_body(cc2, x1, f2w, f2b, w3c, x2o, y3o):
    x2 = (jnp.maximum(cc2[:, :64], 0.0) + _dot(x1[...], f2w[...])
          + f2b[...])
    x2o[...] = x2
    y3o[...] = _dot(x2, w3c[...])


def _l3_body(cc3, x2, f3w, f3b, w4c, x3o, y4o):
    x3 = _dot(x2[...], f3w[...]) + f3b[...] + cc3[:, :64]
    x3o[...] = x3
    y4o[...] = _dot(x3, w4c[...])


def _l4_body(cc4, x3, f4w, f4b, xo):
    xo[...] = _dot(x3[...], f4w[...]) + f4b[...] + cc4[:, :16]


def _row_spec(c):
    return pl.BlockSpec((BR, c), lambda i: (i, 0))


def _full_spec(shape):
    n = len(shape)
    return pl.BlockSpec(shape, lambda i: (0,) * n)


def _tab_spec(c):
    return pl.BlockSpec((4, BR, c), lambda i: (0, i, 0))


def _tc_fused(body, ins, outs):
    return pl.pallas_call(
        body,
        grid=(NP // BR,),
        in_specs=[s for _, s in ins],
        out_specs=[s for _, s in outs],
        out_shape=[jax.ShapeDtypeStruct(sh, jnp.float32) for sh, _ in outs],
    )(*[a for a, _ in ins])


def kernel(dy_positions, dy_feats, box_positions, box_feats, dy_indxs,
           box_indxs, W_cc1, W_cc2, W_cc3, W_cc4,
           fc1_w, fc1_b, fc2_w, fc2_b, fc3_w, fc3_b, fc4_w, fc4_b):
    dyp = jnp.pad(dy_positions, ((0, NP - N), (0, 0)))
    dyf = jnp.pad(dy_feats, ((0, NP - N), (0, 6)))
    bxf = jnp.pad(box_feats, ((0, MPAD - M), (0, 6)))
    dyi = jnp.pad(dy_indxs, ((0, NP - N), (0, 0))).reshape(-1)
    bxi = jnp.pad(box_indxs, ((0, NP - N), (0, 0))).reshape(-1)
    dyx, dyy, dyz = dyp[:, 0], dyp[:, 1], dyp[:, 2]
    bxx, bxy, bxz = (box_positions[:, 0], box_positions[:, 1],
                     box_positions[:, 2])

    w1c = jnp.transpose(W_cc1, (1, 0, 2)).reshape(2, 128)
    w2c = jnp.transpose(W_cc2, (1, 0, 2)).reshape(96, 256)
    w3c = jnp.transpose(W_cc3, (1, 0, 2)).reshape(64, 256)
    w4c = jnp.transpose(jnp.pad(W_cc4, ((0, 0), (0, 0), (0, 29))),
                        (1, 0, 2)).reshape(64, 128)
    f1w = jnp.pad(fc1_w, ((0, 6), (0, 0)))
    f4w = jnp.pad(fc4_w, ((0, 0), (0, 13)))
    f4b = jnp.pad(fc4_b, (0, 13))

    y1d = _tc_y1(dyf, w1c)
    y1b = _tc_y1(bxf, w1c)
    dyo, dyw, ccb = _stage1(dyx, dyy, dyz, bxx, bxy, bxz, dyi, bxi,
                            y1b.reshape(MPAD * 4, 32),
                            y1d.reshape(NP * 4, 32))
    x1, y2 = _tc_fused(
        _l1_body,
        [(ccb, _row_spec(128)), (dyf, _row_spec(8)),
         (f1w, _full_spec((8, 32))),
         (fc1_b.reshape(1, 32), _full_spec((1, 32))),
         (w2c, _full_spec((96, 256)))],
        [((NP, 96), _row_spec(96)), ((NP, 256), _row_spec(256))])

    cc2 = _gather64(y2.reshape(NP * 4, 64), dyo, dyw)
    x2, y3 = _tc_fused(
        _l2_body,
        [(cc2, _row_spec(128)), (x1, _row_spec(96)),
         (fc2_w, _full_spec((96, 64))),
         (fc2_b.reshape(1, 64), _full_spec((1, 64))),
         (w3c, _full_spec((64, 256)))],
        [((NP, 64), _row_spec(64)), ((NP, 256), _row_spec(256))])

    cc3 = _gather64(y3.reshape(NP * 4, 64), dyo, dyw)
    x3, y4 = _tc_fused(
        _l3_body,
        [(cc3, _row_spec(128)), (x2, _row_spec(64)),
         (fc3_w, _full_spec((64, 64))),
         (fc3_b.reshape(1, 64), _full_spec((1, 64))),
         (w4c, _full_spec((64, 128)))],
        [((NP, 64), _row_spec(64)), ((NP, 128), _row_spec(128))])

    cc4 = _gather32(y4.reshape(NP * 4, 32), dyo, dyw)
    (x4,) = _tc_fused(
        _l4_body,
        [(cc4, _row_spec(128)), (x3, _row_spec(64)),
         (f4w, _full_spec((64, 16))),
         (f4b.reshape(1, 16), _full_spec((1, 16)))],
        [((NP, 16), _row_spec(16))])

    return x4[:N, :3]


# speedup vs baseline: 1.1300x; 1.0004x over previous
import functools

import jax
import jax.numpy as jnp
from jax import lax
from jax.experimental import pallas as pl
from jax.experimental.pallas import tpu as pltpu
from jax.experimental.pallas import tpu_sc as plsc

N = 50000
M = 10000
K = 16

NC = 2
NS = 16
NW = NC * NS
L = 16

NPW = 1568
NP = NPW * NW
NPA = 1344
NPB = 1792
PB0 = NS * NPA
CP = 8
EC = CP * K

BR = 512
MPAD = 10240

_SC_PARAMS = pltpu.CompilerParams(needs_layout_passes=False,
                                  use_tc_tiling_on_sc=False)


def _mesh():
    return plsc.VectorSubcoreMesh(core_axis_name="c", subcore_axis_name="s")


def _coord_body(c, ob, sb, tab, qb):
    def body(p, carry):
        ev = ob[pl.ds(p * K, K)]
        raw = ev if c == 0 else jax.lax.shift_right_logical(ev, 2)
        xs = plsc.load_gather(tab, [raw])
        qsplat = plsc.load_gather(qb, [jnp.zeros((K,), jnp.int32) + p])
        rel = xs - qsplat
        r2 = rel * rel
        pos = (rel > 0).astype(jnp.int32)
        if c == 0:
            sb[pl.ds(p * K, K)] = r2
            ob[pl.ds(p * K, K)] = ev * 4 + pos * 2
        elif c == 1:
            sb[pl.ds(p * K, K)] = sb[pl.ds(p * K, K)] + r2
            ob[pl.ds(p * K, K)] = ev + pos
        else:
            sb[pl.ds(p * K, K)] = jnp.exp(-(sb[pl.ds(p * K, K)] + r2))
        return carry
    return body


def _edge_phase(tabs, tlen, qsrcs, idx_in, ebase, pbase, ob, sb, tab, qb,
                npw):
    pltpu.sync_copy(idx_in.at[pl.ds(ebase, npw * K)], ob.at[pl.ds(0, npw * K)])
    for c in range(3):
        pltpu.sync_copy(tabs[c], tab.at[pl.ds(0, tlen)])
        pltpu.sync_copy(qsrcs[c].at[pl.ds(pbase, npw)],
                        qb.at[pl.ds(0, npw)])
        lax.fori_loop(0, npw, _coord_body(c, ob, sb, tab, qb), 0, unroll=4)


BOXSTRIDE = 10240


def _box_phase(btabs, qsrcs, idx_in, ebase, pbase, ob, sb, tab, qb, npw):
    pltpu.sync_copy(idx_in.at[pl.ds(ebase, npw * K)], ob.at[pl.ds(0, npw * K)])
    for t in range(3):
        pltpu.sync_copy(btabs[t], tab.at[pl.ds(t * BOXSTRIDE, M)])
        pltpu.sync_copy(qsrcs[t].at[pl.ds(pbase, npw)],
                        qb.at[pl.ds(t * npw, npw)])

    def body(p, carry):
        ev = ob[pl.ds(p * K, K)]
        iq = jnp.zeros((K,), jnp.int32) + p
        s = None
        bb = None
        for t in range(3):
            xs = plsc.load_gather(tab.at[pl.ds(t * BOXSTRIDE, M)], [ev])
            q = plsc.load_gather(qb.at[pl.ds(t * npw, npw)], [iq])
            rel = xs - q
            r2 = rel * rel
            s = r2 if t == 0 else s + r2
            if t == 0:
                bb = (rel > 0).astype(jnp.int32) * 2
            elif t == 1:
                bb = bb + (rel > 0).astype(jnp.int32)
        sb[pl.ds(p * K, K)] = jnp.exp(-s)
        ob[pl.ds(p * K, K)] = ev * 4 + bb
        return carry

    lax.fori_loop(0, npw, body, 0, unroll=4)


def _gather_sum(ytab, out, offb, wb, rows, outb, gsems, osems, pbase, C,
                nbuf, npw, out_col=0):
    nsub = C // L
    NCH = npw // CP

    def odst(ch):
        return out.at[pl.ds(pbase + ch * CP, CP), pl.ds(out_col, C)]

    def issue(ch, j):
        pltpu.async_copy(
            ytab.at[offb.at[pl.ds(ch * EC, EC)]], rows.at[j], gsems[j])

    for j in range(nbuf):
        issue(j, j)

    def outer(g, carry):
        for j in range(nbuf):
            ch = g * nbuf + j
            pltpu.make_async_copy(
                ytab.at[offb.at[pl.ds(ch * EC, EC)]], rows.at[j],
                gsems[j]).wait()

            @pl.when(ch >= nbuf)
            def _():
                pltpu.make_async_copy(
                    outb.at[j], odst(ch - nbuf), osems[j]).wait()

            def acc_body(p, inner):
                e0 = ch * EC + p * K
                wv = wb[pl.ds(e0, K)]
                for cb in range(nsub):
                    ts = [wv[kk] * rows[j, p * K + kk, pl.ds(cb * L, L)]
                          for kk in range(K)]
                    while len(ts) > 1:
                        ts = [ts[i] + ts[i + 1] for i in range(0, len(ts), 2)]
                    outb[j, p, pl.ds(cb * L, L)] = ts[0]
                return inner

            lax.fori_loop(0, CP, acc_body, 0)

            @pl.when(ch + nbuf < NCH)
            def _():
                issue(ch + nbuf, j)

            pltpu.async_copy(outb.at[j], odst(ch), osems[j])
        return carry

    lax.fori_loop(0, NCH // nbuf, outer, 0)
    for ch in range(NCH - nbuf, NCH):
        pltpu.make_async_copy(
            outb.at[ch % nbuf], odst(ch), osems[ch % nbuf]).wait()


@functools.partial(
    pl.kernel,
    out_type=[
        jax.ShapeDtypeStruct((NP * K,), jnp.int32),
        jax.ShapeDtypeStruct((NP * K,), jnp.float32),
        jax.ShapeDtypeStruct((NP, 128), jnp.float32),
    ],
    mesh=_mesh(),
    scratch_types=[
        pltpu.VMEM((NP,), jnp.float32),
        pltpu.VMEM((3 * NPB,), jnp.float32),
        pltpu.VMEM((NPB * K,), jnp.int32),
        pltpu.VMEM((NPB * K,), jnp.float32),
        pltpu.VMEM((4, EC, 32), jnp.float32),
        pltpu.VMEM((4, CP, 32), jnp.float32),
    ] + [pltpu.SemaphoreType.DMA] * 8,
    compiler_params=_SC_PARAMS,
)
def _stage1(dyx, dyy, dyz, bxx, bxy, bxz, dyi, bxi, y1b, y1d,
            dyo, dyw, ccb,
            tab, qb, ob, sb, rows, outb,
            g0, g1, g2, g3, o0, o1, o2, o3):
    cc = lax.axis_index("c")
    ss = lax.axis_index("s")
    gsems = (g0, g1, g2, g3)
    osems = (o0, o1, o2, o3)
    qsrcs = (dyx, dyy, dyz)

    def run(npw, pbase):
        ebase = pbase * K
        _box_phase((bxx, bxy, bxz), qsrcs, bxi, ebase, pbase, ob, sb, tab,
                   qb, npw)
        _gather_sum(y1b, ccb, ob, sb, rows, outb, gsems, osems, pbase, 32,
                    nbuf=4, npw=npw, out_col=0)
        _edge_phase(qsrcs, NP, qsrcs, dyi, ebase, pbase, ob, sb, tab, qb,
                    npw)
        pltpu.sync_copy(ob.at[pl.ds(0, npw * K)], dyo.at[pl.ds(ebase, npw * K)])
        pltpu.sync_copy(sb.at[pl.ds(0, npw * K)], dyw.at[pl.ds(ebase, npw * K)])
        _gather_sum(y1d, ccb, ob, sb, rows, outb, gsems, osems, pbase, 32,
                    nbuf=4, npw=npw, out_col=32)

    @pl.when(cc == 0)
    def _():
        run(NPA, ss * NPA)

    @pl.when(cc == 1)
    def _():
        run(NPB, PB0 + ss * NPB)


GNBUF = 7


def _make_gather(C):
    @functools.partial(
        pl.kernel,
        out_type=jax.ShapeDtypeStruct((NP, 128), jnp.float32),
        mesh=_mesh(),
        scratch_types=[
            pltpu.VMEM((NPB * K,), jnp.int32),
            pltpu.VMEM((NPB * K,), jnp.float32),
            pltpu.VMEM((GNBUF, EC, C), jnp.float32),
            pltpu.VMEM((GNBUF, CP, C), jnp.float32),
        ] + [pltpu.SemaphoreType.DMA] * (2 * GNBUF),
        compiler_params=_SC_PARAMS,
    )
    def k(ytab, off, w, out, offb, wb, rows, outb,
          g0, g1, g2, g3, g4, g5, g6, o0, o1, o2, o3, o4, o5, o6):
        cc = lax.axis_index("c")
        ss = lax.axis_index("s")

        def run(npw, pbase):
            ebase = pbase * K
            pltpu.sync_copy(off.at[pl.ds(ebase, npw * K)],
                            offb.at[pl.ds(0, npw * K)])
            pltpu.sync_copy(w.at[pl.ds(ebase, npw * K)],
                            wb.at[pl.ds(0, npw * K)])
            _gather_sum(ytab, out, offb, wb, rows, outb,
                        (g0, g1, g2, g3, g4, g5, g6),
                        (o0, o1, o2, o3, o4, o5, o6), pbase, C,
                        nbuf=GNBUF, npw=npw)

        @pl.when(cc == 0)
        def _():
            run(NPA, ss * NPA)

        @pl.when(cc == 1)
        def _():
            run(NPB, PB0 + ss * NPB)

    return k


_gather64 = _make_gather(64)
_gather32 = _make_gather(32)


def _dot(a, b):
    return jax.lax.dot_general(
        a, b, (((1,), (0,)), ((), ())),
        preferred_element_type=jnp.float32)


def _y1_body(ft, w1c, o):
    f = ft[...]
    w = w1c[...]
    o[...] = f[:, 0:1] * w[0:1, :] + f[:, 1:2] * w[1:2, :]


def _tc_y1(x, w1c):
    R = x.shape[0]
    return pl.pallas_call(
        _y1_body,
        grid=(R // BR,),
        in_specs=[pl.BlockSpec((BR, 8), lambda i: (i, 0)),
                  pl.BlockSpec((2, 128), lambda i: (0, 0))],
        out_specs=pl.BlockSpec((BR, 128), lambda i: (i, 0)),
        out_shape=jax.ShapeDtypeStruct((R, 128), jnp.float32),
    )(x, w1c)


def _l1_body(ccb, ft, f1w, f1b, w2c, x1o, y2o):
    self1 = _dot(ft[...], f1w[...]) + f1b[...]
    x1 = jnp.maximum(
        jnp.concatenate([ccb[:, :64], self1], axis=1), 0.0)
    x1o[...] = x1
    y2o[...] = _dot(x1, w2c[...])


def _l2_body(cc2, x1, f2w, f2b, w3c, x2o, y3o):
    x2 = (jnp.maximum(cc2[:, :64], 0.0) + _dot(x1[...], f2w[...])
          + f2b[...])
    x2o[...] = x2
    y3o[...] = _dot(x2, w3c[...])


def _l3_body(cc3, x2, f3w, f3b, w4c, x3o, y4o):
    x3 = _dot(x2[...], f3w[...]) + f3b[...] + cc3[:, :64]
    x3o[...] = x3
    y4o[...] = _dot(x3, w4c[...])


def _l4_body(cc4, x3, f4w, f4b, xo):
    xo[...] = _dot(x3[...], f4w[...]) + f4b[...] + cc4[:, :16]


def _row_spec(c):
    return pl.BlockSpec((BR, c), lambda i: (i, 0))


def _full_spec(shape):
    n = len(shape)
    return pl.BlockSpec(shape, lambda i: (0,) * n)


def _tab_spec(c):
    return pl.BlockSpec((4, BR, c), lambda i: (0, i, 0))


def _tc_fused(body, ins, outs):
    return pl.pallas_call(
        body,
        grid=(NP // BR,),
        in_specs=[s for _, s in ins],
        out_specs=[s for _, s in outs],
        out_shape=[jax.ShapeDtypeStruct(sh, jnp.float32) for sh, _ in outs],
    )(*[a for a, _ in ins])


def kernel(dy_positions, dy_feats, box_positions, box_feats, dy_indxs,
           box_indxs, W_cc1, W_cc2, W_cc3, W_cc4,
           fc1_w, fc1_b, fc2_w, fc2_b, fc3_w, fc3_b, fc4_w, fc4_b):
    dyp = jnp.pad(dy_positions, ((0, NP - N), (0, 0)))
    dyf = jnp.pad(dy_feats, ((0, NP - N), (0, 6)))
    bxf = jnp.pad(box_feats, ((0, MPAD - M), (0, 6)))
    dyi = jnp.pad(dy_indxs, ((0, NP - N), (0, 0))).reshape(-1)
    bxi = jnp.pad(box_indxs, ((0, NP - N), (0, 0))).reshape(-1)
    dyx, dyy, dyz = dyp[:, 0], dyp[:, 1], dyp[:, 2]
    bxx, bxy, bxz = (box_positions[:, 0], box_positions[:, 1],
                     box_positions[:, 2])

    w1c = jnp.transpose(W_cc1, (1, 0, 2)).reshape(2, 128)
    w2c = jnp.transpose(W_cc2, (1, 0, 2)).reshape(96, 256)
    w3c = jnp.transpose(W_cc3, (1, 0, 2)).reshape(64, 256)
    w4c = jnp.transpose(jnp.pad(W_cc4, ((0, 0), (0, 0), (0, 29))),
                        (1, 0, 2)).reshape(64, 128)
    f1w = jnp.pad(fc1_w, ((0, 6), (0, 0)))
    f4w = jnp.pad(fc4_w, ((0, 0), (0, 13)))
    f4b = jnp.pad(fc4_b, (0, 13))

    y1d = _tc_y1(dyf, w1c)
    y1b = _tc_y1(bxf, w1c)
    dyo, dyw, ccb = _stage1(dyx, dyy, dyz, bxx, bxy, bxz, dyi, bxi,
                            y1b.reshape(MPAD * 4, 32),
                            y1d.reshape(NP * 4, 32))
    x1, y2 = _tc_fused(
        _l1_body,
        [(ccb, _row_spec(128)), (dyf, _row_spec(8)),
         (f1w, _full_spec((8, 32))),
         (fc1_b.reshape(1, 32), _full_spec((1, 32))),
         (w2c, _full_spec((96, 256)))],
        [((NP, 96), _row_spec(96)), ((NP, 256), _row_spec(256))])

    cc2 = _gather64(y2.reshape(NP * 4, 64), dyo, dyw)
    x2, y3 = _tc_fused(
        _l2_body,
        [(cc2, _row_spec(128)), (x1, _row_spec(96)),
         (fc2_w, _full_spec((96, 64))),
         (fc2_b.reshape(1, 64), _full_spec((1, 64))),
         (w3c, _full_spec((64, 256)))],
        [((NP, 64), _row_spec(64)), ((NP, 256), _row_spec(256))])

    cc3 = _gather64(y3.reshape(NP * 4, 64), dyo, dyw)
    x3, y4 = _tc_fused(
        _l3_body,
        [(cc3, _row_spec(128)), (x2, _row_spec(64)),
         (fc3_w, _full_spec((64, 64))),
         (fc3_b.reshape(1, 64), _full_spec((1, 64))),
         (w4c, _full_spec((64, 128)))],
        [((NP, 64), _row_spec(64)), ((NP, 128), _row_spec(128))])

    cc4 = _gather32(y4.reshape(NP * 4, 32), dyo, dyw)
    (x4,) = _tc_fused(
        _l4_body,
        [(cc4, _row_spec(128)), (x3, _row_spec(64)),
         (f4w, _full_spec((64, 16))),
         (f4b.reshape(1, 16), _full_spec((1, 16)))],
        [((NP, 16), _row_spec(16))])

    return x4[:N, :3]


# speedup vs baseline: 1.2574x; 1.1127x over previous
import functools

import jax
import jax.numpy as jnp
from jax import lax
from jax.experimental import pallas as pl
from jax.experimental.pallas import tpu as pltpu
from jax.experimental.pallas import tpu_sc as plsc

N = 50000
M = 10000
K = 16

NC = 2
NS = 16
NW = NC * NS
L = 16

NPW = 1568
NP = NPW * NW
NPA = 1568
NPB = 1568
PB0 = NS * NPA
CP = 8
EC = CP * K

BR = 512
MPAD = 10240

_SC_PARAMS = pltpu.CompilerParams(needs_layout_passes=False,
                                  use_tc_tiling_on_sc=False)


def _mesh():
    return plsc.VectorSubcoreMesh(core_axis_name="c", subcore_axis_name="s")


def _coord_body(c, ob, sb, tab, qb):
    def body(p, carry):
        ev = ob[pl.ds(p * K, K)]
        raw = ev if c == 0 else jax.lax.shift_right_logical(ev, 2)
        xs = plsc.load_gather(tab, [raw])
        qsplat = plsc.load_gather(qb, [jnp.zeros((K,), jnp.int32) + p])
        rel = xs - qsplat
        r2 = rel * rel
        pos = (rel > 0).astype(jnp.int32)
        if c == 0:
            sb[pl.ds(p * K, K)] = r2
            ob[pl.ds(p * K, K)] = ev * 4 + pos * 2
        elif c == 1:
            sb[pl.ds(p * K, K)] = sb[pl.ds(p * K, K)] + r2
            ob[pl.ds(p * K, K)] = ev + pos
        else:
            sb[pl.ds(p * K, K)] = jnp.exp(-(sb[pl.ds(p * K, K)] + r2))
        return carry
    return body


def _edge_phase(tabs, tlen, qsrcs, idx_in, ebase, pbase, ob, sb, tab, qb,
                npw):
    pltpu.sync_copy(idx_in.at[pl.ds(ebase, npw * K)], ob.at[pl.ds(0, npw * K)])
    for c in range(3):
        pltpu.sync_copy(tabs[c], tab.at[pl.ds(0, tlen)])
        pltpu.sync_copy(qsrcs[c].at[pl.ds(pbase, npw)],
                        qb.at[pl.ds(0, npw)])
        lax.fori_loop(0, npw, _coord_body(c, ob, sb, tab, qb), 0, unroll=4)


BOXSTRIDE = 10240


def _box_phase(btabs, qsrcs, idx_in, ebase, pbase, ob, sb, tab, qb, npw):
    pltpu.sync_copy(idx_in.at[pl.ds(ebase, npw * K)], ob.at[pl.ds(0, npw * K)])
    for t in range(3):
        pltpu.sync_copy(btabs[t], tab.at[pl.ds(t * BOXSTRIDE, M)])
        pltpu.sync_copy(qsrcs[t].at[pl.ds(pbase, npw)],
                        qb.at[pl.ds(t * npw, npw)])

    def body(p, carry):
        ev = ob[pl.ds(p * K, K)]
        iq = jnp.zeros((K,), jnp.int32) + p
        s = None
        bb = None
        for t in range(3):
            xs = plsc.load_gather(tab.at[pl.ds(t * BOXSTRIDE, M)], [ev])
            q = plsc.load_gather(qb.at[pl.ds(t * npw, npw)], [iq])
            rel = xs - q
            r2 = rel * rel
            s = r2 if t == 0 else s + r2
            if t == 0:
                bb = (rel > 0).astype(jnp.int32) * 2
            elif t == 1:
                bb = bb + (rel > 0).astype(jnp.int32)
        sb[pl.ds(p * K, K)] = jnp.exp(-s)
        ob[pl.ds(p * K, K)] = ev * 4 + bb
        return carry

    lax.fori_loop(0, npw, body, 0, unroll=4)


def _gather_sum(ytab, out, offb, wb, rows, outb, gsems, osems, pbase, C,
                nbuf, npw, out_col=0):
    nsub = C // 32
    NCH = npw // CP

    def odst(ch):
        return out.at[pl.ds(pbase + ch * CP, CP), pl.ds(out_col, C)]

    def issue(ch, j):
        pltpu.async_copy(
            ytab.at[offb.at[pl.ds(ch * EC, EC)]], rows.at[j], gsems[j])

    for j in range(nbuf):
        issue(j, j)

    def outer(g, carry):
        for j in range(nbuf):
            ch = g * nbuf + j
            pltpu.make_async_copy(
                ytab.at[offb.at[pl.ds(ch * EC, EC)]], rows.at[j],
                gsems[j]).wait()

            @pl.when(ch >= nbuf)
            def _():
                pltpu.make_async_copy(
                    outb.at[j], odst(ch - nbuf), osems[j]).wait()

            def acc_body(p, inner):
                e0 = ch * EC + p * K
                wv = wb[pl.ds(e0, K)]
                for cb in range(nsub):
                    ts_e = []
                    ts_o = []
                    for kk in range(K):
                        vi = plsc.bitcast(
                            rows[j, p * K + kk, pl.ds(cb * 32, 32)],
                            jnp.int32)
                        ev = plsc.bitcast(vi << 16, jnp.float32)
                        ov = plsc.bitcast(vi & jnp.int32(-65536),
                                          jnp.float32)
                        ts_e.append(wv[kk] * ev)
                        ts_o.append(wv[kk] * ov)
                    while len(ts_e) > 1:
                        ts_e = [ts_e[i] + ts_e[i + 1]
                                for i in range(0, len(ts_e), 2)]
                        ts_o = [ts_o[i] + ts_o[i + 1]
                                for i in range(0, len(ts_o), 2)]
                    outb[j, p, pl.ds(cb * 32, L)] = ts_e[0]
                    outb[j, p, pl.ds(cb * 32 + L, L)] = ts_o[0]
                return inner

            lax.fori_loop(0, CP, acc_body, 0)

            @pl.when(ch + nbuf < NCH)
            def _():
                issue(ch + nbuf, j)

            pltpu.async_copy(outb.at[j], odst(ch), osems[j])
        return carry

    lax.fori_loop(0, NCH // nbuf, outer, 0)
    for ch in range(NCH - nbuf, NCH):
        pltpu.make_async_copy(
            outb.at[ch % nbuf], odst(ch), osems[ch % nbuf]).wait()


@functools.partial(
    pl.kernel,
    out_type=[
        jax.ShapeDtypeStruct((NP * K,), jnp.int32),
        jax.ShapeDtypeStruct((NP * K,), jnp.float32),
        jax.ShapeDtypeStruct((NP, 128), jnp.float32),
    ],
    mesh=_mesh(),
    scratch_types=[
        pltpu.VMEM((NP,), jnp.float32),
        pltpu.VMEM((3 * NPB,), jnp.float32),
        pltpu.VMEM((NPB * K,), jnp.int32),
        pltpu.VMEM((NPB * K,), jnp.float32),
        pltpu.VMEM((4, EC, 32), jnp.bfloat16),
        pltpu.VMEM((4, CP, 32), jnp.float32),
    ] + [pltpu.SemaphoreType.DMA] * 8,
    compiler_params=_SC_PARAMS,
)
def _stage1(dyx, dyy, dyz, bxx, bxy, bxz, dyi, bxi, y1b, y1d,
            dyo, dyw, ccb,
            tab, qb, ob, sb, rows, outb,
            g0, g1, g2, g3, o0, o1, o2, o3):
    cc = lax.axis_index("c")
    ss = lax.axis_index("s")
    gsems = (g0, g1, g2, g3)
    osems = (o0, o1, o2, o3)
    qsrcs = (dyx, dyy, dyz)

    def run(npw, pbase):
        ebase = pbase * K
        _box_phase((bxx, bxy, bxz), qsrcs, bxi, ebase, pbase, ob, sb, tab,
                   qb, npw)
        _gather_sum(y1b, ccb, ob, sb, rows, outb, gsems, osems, pbase, 32,
                    nbuf=4, npw=npw, out_col=0)
        _edge_phase(qsrcs, NP, qsrcs, dyi, ebase, pbase, ob, sb, tab, qb,
                    npw)
        pltpu.sync_copy(ob.at[pl.ds(0, npw * K)], dyo.at[pl.ds(ebase, npw * K)])
        pltpu.sync_copy(sb.at[pl.ds(0, npw * K)], dyw.at[pl.ds(ebase, npw * K)])
        _gather_sum(y1d, ccb, ob, sb, rows, outb, gsems, osems, pbase, 32,
                    nbuf=4, npw=npw, out_col=32)

    @pl.when(cc == 0)
    def _():
        run(NPA, ss * NPA)

    @pl.when(cc == 1)
    def _():
        run(NPB, PB0 + ss * NPB)


GNBUF = 7


def _make_gather(C):
    @functools.partial(
        pl.kernel,
        out_type=jax.ShapeDtypeStruct((NP, 128), jnp.float32),
        mesh=_mesh(),
        scratch_types=[
            pltpu.VMEM((NPB * K,), jnp.int32),
            pltpu.VMEM((NPB * K,), jnp.float32),
            pltpu.VMEM((GNBUF, EC, C), jnp.bfloat16),
            pltpu.VMEM((GNBUF, CP, C), jnp.float32),
        ] + [pltpu.SemaphoreType.DMA] * (2 * GNBUF),
        compiler_params=_SC_PARAMS,
    )
    def k(ytab, off, w, out, offb, wb, rows, outb,
          g0, g1, g2, g3, g4, g5, g6, o0, o1, o2, o3, o4, o5, o6):
        cc = lax.axis_index("c")
        ss = lax.axis_index("s")

        def run(npw, pbase):
            ebase = pbase * K
            pltpu.sync_copy(off.at[pl.ds(ebase, npw * K)],
                            offb.at[pl.ds(0, npw * K)])
            pltpu.sync_copy(w.at[pl.ds(ebase, npw * K)],
                            wb.at[pl.ds(0, npw * K)])
            _gather_sum(ytab, out, offb, wb, rows, outb,
                        (g0, g1, g2, g3, g4, g5, g6),
                        (o0, o1, o2, o3, o4, o5, o6), pbase, C,
                        nbuf=GNBUF, npw=npw)

        @pl.when(cc == 0)
        def _():
            run(NPA, ss * NPA)

        @pl.when(cc == 1)
        def _():
            run(NPB, PB0 + ss * NPB)

    return k


_gather64 = _make_gather(64)
_gather32 = _make_gather(32)


def _dot(a, b):
    return jax.lax.dot_general(
        a, b, (((1,), (0,)), ((), ())),
        preferred_element_type=jnp.float32)


def _y1_body(ft, w1c, o):
    f = ft[...]
    w = w1c[...]
    o[...] = (f[:, 0:1] * w[0:1, :]
              + f[:, 1:2] * w[1:2, :]).astype(jnp.bfloat16)


def _tc_y1(x, w1c):
    R = x.shape[0]
    return pl.pallas_call(
        _y1_body,
        grid=(R // BR,),
        in_specs=[pl.BlockSpec((BR, 8), lambda i: (i, 0)),
                  pl.BlockSpec((2, 128), lambda i: (0, 0))],
        out_specs=pl.BlockSpec((BR, 128), lambda i: (i, 0)),
        out_shape=jax.ShapeDtypeStruct((R, 128), jnp.bfloat16),
    )(x, w1c)


def _l1_body(ccb, ft, f1w, f1b, w2c, x1o, y2o):
    self1 = _dot(ft[...], f1w[...]) + f1b[...]
    x1 = jnp.maximum(
        jnp.concatenate([ccb[:, :64], self1], axis=1), 0.0)
    x1o[...] = x1
    y2o[...] = _dot(x1, w2c[...]).astype(jnp.bfloat16)


def _l2_body(cc2, x1, f2w, f2b, w3c, x2o, y3o):
    x2 = (jnp.maximum(cc2[:, :64], 0.0) + _dot(x1[...], f2w[...])
          + f2b[...])
    x2o[...] = x2
    y3o[...] = _dot(x2, w3c[...]).astype(jnp.bfloat16)


def _l3_body(cc3, x2, f3w, f3b, w4c, x3o, y4o):
    x3 = _dot(x2[...], f3w[...]) + f3b[...] + cc3[:, :64]
    x3o[...] = x3
    y4o[...] = _dot(x3, w4c[...]).astype(jnp.bfloat16)


def _l4_body(cc4, x3, f4w, f4b, xo):
    xo[...] = _dot(x3[...], f4w[...]) + f4b[...] + cc4[:, :16]


def _row_spec(c):
    return pl.BlockSpec((BR, c), lambda i: (i, 0))


def _full_spec(shape):
    n = len(shape)
    return pl.BlockSpec(shape, lambda i: (0,) * n)


def _tab_spec(c):
    return pl.BlockSpec((4, BR, c), lambda i: (0, i, 0))


PERM32 = [(i // 2) + 16 * (i % 2) for i in range(32)]


def _interleave_cols(w):
    C = w.shape[-1]
    perm = [b * 32 + PERM32[i] for b in range(C // 32) for i in range(32)]
    return w[:, perm]


def _tc_fused(body, ins, outs):
    return pl.pallas_call(
        body,
        grid=(NP // BR,),
        in_specs=[s for _, s in ins],
        out_specs=[s for _, s, _ in outs],
        out_shape=[jax.ShapeDtypeStruct(sh, dt) for sh, _, dt in outs],
    )(*[a for a, _ in ins])


def kernel(dy_positions, dy_feats, box_positions, box_feats, dy_indxs,
           box_indxs, W_cc1, W_cc2, W_cc3, W_cc4,
           fc1_w, fc1_b, fc2_w, fc2_b, fc3_w, fc3_b, fc4_w, fc4_b):
    dyp = jnp.pad(dy_positions, ((0, NP - N), (0, 0)))
    dyf = jnp.pad(dy_feats, ((0, NP - N), (0, 6)))
    bxf = jnp.pad(box_feats, ((0, MPAD - M), (0, 6)))
    dyi = jnp.pad(dy_indxs, ((0, NP - N), (0, 0))).reshape(-1)
    bxi = jnp.pad(box_indxs, ((0, NP - N), (0, 0))).reshape(-1)
    dyx, dyy, dyz = dyp[:, 0], dyp[:, 1], dyp[:, 2]
    bxx, bxy, bxz = (box_positions[:, 0], box_positions[:, 1],
                     box_positions[:, 2])

    w1c = _interleave_cols(
        jnp.transpose(W_cc1, (1, 0, 2)).reshape(2, 128))
    w2c = _interleave_cols(
        jnp.transpose(W_cc2, (1, 0, 2)).reshape(96, 256))
    w3c = _interleave_cols(
        jnp.transpose(W_cc3, (1, 0, 2)).reshape(64, 256))
    w4c = _interleave_cols(
        jnp.transpose(jnp.pad(W_cc4, ((0, 0), (0, 0), (0, 29))),
                      (1, 0, 2)).reshape(64, 128))
    f1w = jnp.pad(fc1_w, ((0, 6), (0, 0)))
    f4w = jnp.pad(fc4_w, ((0, 0), (0, 13)))
    f4b = jnp.pad(fc4_b, (0, 13))

    y1d = _tc_y1(dyf, w1c)
    y1b = _tc_y1(bxf, w1c)
    dyo, dyw, ccb = _stage1(dyx, dyy, dyz, bxx, bxy, bxz, dyi, bxi,
                            y1b.reshape(MPAD * 4, 32),
                            y1d.reshape(NP * 4, 32))
    x1, y2 = _tc_fused(
        _l1_body,
        [(ccb, _row_spec(128)), (dyf, _row_spec(8)),
         (f1w, _full_spec((8, 32))),
         (fc1_b.reshape(1, 32), _full_spec((1, 32))),
         (w2c, _full_spec((96, 256)))],
        [((NP, 96), _row_spec(96), jnp.float32),
         ((NP, 256), _row_spec(256), jnp.bfloat16)])

    cc2 = _gather64(y2.reshape(NP * 4, 64), dyo, dyw)
    x2, y3 = _tc_fused(
        _l2_body,
        [(cc2, _row_spec(128)), (x1, _row_spec(96)),
         (fc2_w, _full_spec((96, 64))),
         (fc2_b.reshape(1, 64), _full_spec((1, 64))),
         (w3c, _full_spec((64, 256)))],
        [((NP, 64), _row_spec(64), jnp.float32),
         ((NP, 256), _row_spec(256), jnp.bfloat16)])

    cc3 = _gather64(y3.reshape(NP * 4, 64), dyo, dyw)
    x3, y4 = _tc_fused(
        _l3_body,
        [(cc3, _row_spec(128)), (x2, _row_spec(64)),
         (fc3_w, _full_spec((64, 64))),
         (fc3_b.reshape(1, 64), _full_spec((1, 64))),
         (w4c, _full_spec((64, 128)))],
        [((NP, 64), _row_spec(64), jnp.float32),
         ((NP, 128), _row_spec(128), jnp.bfloat16)])

    cc4 = _gather32(y4.reshape(NP * 4, 32), dyo, dyw)
    (x4,) = _tc_fused(
        _l4_body,
        [(cc4, _row_spec(128)), (x3, _row_spec(64)),
         (f4w, _full_spec((64, 16))),
         (f4b.reshape(1, 16), _full_spec((1, 16)))],
        [((NP, 16), _row_spec(16), jnp.float32)])

    return x4[:N, :3]


# speedup vs baseline: 1.2619x; 1.0036x over previous
import functools

import jax
import jax.numpy as jnp
from jax import lax
from jax.experimental import pallas as pl
from jax.experimental.pallas import tpu as pltpu
from jax.experimental.pallas import tpu_sc as plsc

N = 50000
M = 10000
K = 16

NC = 2
NS = 16
NW = NC * NS
L = 16

NPW = 1568
NP = NPW * NW
NPA = 1568
NPB = 1568
PB0 = NS * NPA
CP = 8
EC = CP * K

BR = 512
MPAD = 10240

_SC_PARAMS = pltpu.CompilerParams(needs_layout_passes=False,
                                  use_tc_tiling_on_sc=False)


def _mesh():
    return plsc.VectorSubcoreMesh(core_axis_name="c", subcore_axis_name="s")


def _coord_body(c, ob, sb, tab, qb):
    def body(p, carry):
        ev = ob[pl.ds(p * K, K)]
        raw = ev if c == 0 else jax.lax.shift_right_logical(ev, 2)
        xs = plsc.load_gather(tab, [raw])
        qsplat = plsc.load_gather(qb, [jnp.zeros((K,), jnp.int32) + p])
        rel = xs - qsplat
        r2 = rel * rel
        pos = (rel > 0).astype(jnp.int32)
        if c == 0:
            sb[pl.ds(p * K, K)] = r2
            ob[pl.ds(p * K, K)] = ev * 4 + pos * 2
        elif c == 1:
            sb[pl.ds(p * K, K)] = sb[pl.ds(p * K, K)] + r2
            ob[pl.ds(p * K, K)] = ev + pos
        else:
            sb[pl.ds(p * K, K)] = jnp.exp(-(sb[pl.ds(p * K, K)] + r2))
        return carry
    return body


def _edge_phase(tabs, tlen, qsrcs, idx_in, ebase, pbase, ob, sb, tab, qb,
                npw):
    pltpu.sync_copy(idx_in.at[pl.ds(ebase, npw * K)], ob.at[pl.ds(0, npw * K)])
    for c in range(3):
        pltpu.sync_copy(tabs[c], tab.at[pl.ds(0, tlen)])
        pltpu.sync_copy(qsrcs[c].at[pl.ds(pbase, npw)],
                        qb.at[pl.ds(0, npw)])
        lax.fori_loop(0, npw, _coord_body(c, ob, sb, tab, qb), 0, unroll=4)


BOXSTRIDE = 10240


def _box_phase(btabs, qsrcs, idx_in, ebase, pbase, ob, sb, tab, qb, npw):
    pltpu.sync_copy(idx_in.at[pl.ds(ebase, npw * K)], ob.at[pl.ds(0, npw * K)])
    for t in range(3):
        pltpu.sync_copy(btabs[t], tab.at[pl.ds(t * BOXSTRIDE, M)])
        pltpu.sync_copy(qsrcs[t].at[pl.ds(pbase, npw)],
                        qb.at[pl.ds(t * npw, npw)])

    def body(p, carry):
        ev = ob[pl.ds(p * K, K)]
        iq = jnp.zeros((K,), jnp.int32) + p
        s = None
        bb = None
        for t in range(3):
            xs = plsc.load_gather(tab.at[pl.ds(t * BOXSTRIDE, M)], [ev])
            q = plsc.load_gather(qb.at[pl.ds(t * npw, npw)], [iq])
            rel = xs - q
            r2 = rel * rel
            s = r2 if t == 0 else s + r2
            if t == 0:
                bb = (rel > 0).astype(jnp.int32) * 2
            elif t == 1:
                bb = bb + (rel > 0).astype(jnp.int32)
        sb[pl.ds(p * K, K)] = jnp.exp(-s)
        ob[pl.ds(p * K, K)] = ev * 4 + bb
        return carry

    lax.fori_loop(0, npw, body, 0, unroll=4)


def _gather_sum(ytab, out, offb, wb, rows, outb, gsems, osems, pbase, C,
                nbuf, npw, out_col=0):
    nsub = C // 32
    NCH = npw // CP

    def odst(ch):
        return out.at[pl.ds(pbase + ch * CP, CP), pl.ds(out_col, C)]

    def issue(ch, j):
        pltpu.async_copy(
            ytab.at[offb.at[pl.ds(ch * EC, EC)]], rows.at[j], gsems[j])

    for j in range(nbuf):
        issue(j, j)

    def outer(g, carry):
        for j in range(nbuf):
            ch = g * nbuf + j
            pltpu.make_async_copy(
                ytab.at[offb.at[pl.ds(ch * EC, EC)]], rows.at[j],
                gsems[j]).wait()

            @pl.when(ch >= nbuf)
            def _():
                pltpu.make_async_copy(
                    outb.at[j], odst(ch - nbuf), osems[j]).wait()

            def acc_body(p, inner):
                e0 = ch * EC + p * K
                wv = wb[pl.ds(e0, K)]
                for cb in range(nsub):
                    ts_e = []
                    ts_o = []
                    for kk in range(K):
                        vi = plsc.bitcast(
                            rows[j, p * K + kk, pl.ds(cb * 32, 32)],
                            jnp.int32)
                        ev = plsc.bitcast(vi << 16, jnp.float32)
                        ov = plsc.bitcast(vi & jnp.int32(-65536),
                                          jnp.float32)
                        ts_e.append(wv[kk] * ev)
                        ts_o.append(wv[kk] * ov)
                    while len(ts_e) > 1:
                        ts_e = [ts_e[i] + ts_e[i + 1]
                                for i in range(0, len(ts_e), 2)]
                        ts_o = [ts_o[i] + ts_o[i + 1]
                                for i in range(0, len(ts_o), 2)]
                    outb[j, p, pl.ds(cb * 32, L)] = ts_e[0]
                    outb[j, p, pl.ds(cb * 32 + L, L)] = ts_o[0]
                return inner

            lax.fori_loop(0, CP, acc_body, 0)

            @pl.when(ch + nbuf < NCH)
            def _():
                issue(ch + nbuf, j)

            pltpu.async_copy(outb.at[j], odst(ch), osems[j])
        return carry

    lax.fori_loop(0, NCH // nbuf, outer, 0)
    for ch in range(NCH - nbuf, NCH):
        pltpu.make_async_copy(
            outb.at[ch % nbuf], odst(ch), osems[ch % nbuf]).wait()


@functools.partial(
    pl.kernel,
    out_type=[
        jax.ShapeDtypeStruct((NP * K,), jnp.int32),
        jax.ShapeDtypeStruct((NP * K,), jnp.float32),
        jax.ShapeDtypeStruct((NP, 128), jnp.float32),
    ],
    mesh=_mesh(),
    scratch_types=[
        pltpu.VMEM((NP,), jnp.float32),
        pltpu.VMEM((3 * NPB,), jnp.float32),
        pltpu.VMEM((NPB * K,), jnp.int32),
        pltpu.VMEM((NPB * K,), jnp.float32),
        pltpu.VMEM((4, EC, 32), jnp.bfloat16),
        pltpu.VMEM((4, CP, 32), jnp.float32),
    ] + [pltpu.SemaphoreType.DMA] * 8,
    compiler_params=_SC_PARAMS,
)
def _stage1(dyx, dyy, dyz, bxx, bxy, bxz, dyi, bxi, y1b, y1d,
            dyo, dyw, ccb,
            tab, qb, ob, sb, rows, outb,
            g0, g1, g2, g3, o0, o1, o2, o3):
    cc = lax.axis_index("c")
    ss = lax.axis_index("s")
    gsems = (g0, g1, g2, g3)
    osems = (o0, o1, o2, o3)
    qsrcs = (dyx, dyy, dyz)

    def run(npw, pbase):
        ebase = pbase * K
        _box_phase((bxx, bxy, bxz), qsrcs, bxi, ebase, pbase, ob, sb, tab,
                   qb, npw)
        _gather_sum(y1b, ccb, ob, sb, rows, outb, gsems, osems, pbase, 32,
                    nbuf=4, npw=npw, out_col=0)
        _edge_phase(qsrcs, NP, qsrcs, dyi, ebase, pbase, ob, sb, tab, qb,
                    npw)
        pltpu.sync_copy(ob.at[pl.ds(0, npw * K)], dyo.at[pl.ds(ebase, npw * K)])
        pltpu.sync_copy(sb.at[pl.ds(0, npw * K)], dyw.at[pl.ds(ebase, npw * K)])
        _gather_sum(y1d, ccb, ob, sb, rows, outb, gsems, osems, pbase, 32,
                    nbuf=4, npw=npw, out_col=32)

    @pl.when(cc == 0)
    def _():
        run(NPA, ss * NPA)

    @pl.when(cc == 1)
    def _():
        run(NPB, PB0 + ss * NPB)


GNBUF = 7


def _make_gather(C):
    @functools.partial(
        pl.kernel,
        out_type=jax.ShapeDtypeStruct((NP, 128), jnp.float32),
        mesh=_mesh(),
        scratch_types=[
            pltpu.VMEM((NPB * K,), jnp.int32),
            pltpu.VMEM((NPB * K,), jnp.float32),
            pltpu.VMEM((GNBUF, EC, C), jnp.bfloat16),
            pltpu.VMEM((GNBUF, CP, C), jnp.float32),
        ] + [pltpu.SemaphoreType.DMA] * (2 * GNBUF),
        compiler_params=_SC_PARAMS,
    )
    def k(ytab, off, w, out, offb, wb, rows, outb,
          g0, g1, g2, g3, g4, g5, g6, o0, o1, o2, o3, o4, o5, o6):
        cc = lax.axis_index("c")
        ss = lax.axis_index("s")

        def run(npw, pbase):
            ebase = pbase * K
            pltpu.sync_copy(off.at[pl.ds(ebase, npw * K)],
                            offb.at[pl.ds(0, npw * K)])
            pltpu.sync_copy(w.at[pl.ds(ebase, npw * K)],
                            wb.at[pl.ds(0, npw * K)])
            _gather_sum(ytab, out, offb, wb, rows, outb,
                        (g0, g1, g2, g3, g4, g5, g6),
                        (o0, o1, o2, o3, o4, o5, o6), pbase, C,
                        nbuf=GNBUF, npw=npw)

        @pl.when(cc == 0)
        def _():
            run(NPA, ss * NPA)

        @pl.when(cc == 1)
        def _():
            run(NPB, PB0 + ss * NPB)

    return k


_gather64 = _make_gather(64)
_gather32 = _make_gather(32)


def _dot(a, b):
    return jax.lax.dot_general(
        a, b, (((1,), (0,)), ((), ())),
        preferred_element_type=jnp.float32)


def _y1_body(ft, w1c, o):
    o[...] = _dot(ft[...], w1c[...]).astype(jnp.bfloat16)


def _tc_y1(x, w1c):
    R = x.shape[0]
    return pl.pallas_call(
        _y1_body,
        grid=(R // BR,),
        in_specs=[pl.BlockSpec((BR, 8), lambda i: (i, 0)),
                  pl.BlockSpec((8, 128), lambda i: (0, 0))],
        out_specs=pl.BlockSpec((BR, 128), lambda i: (i, 0)),
        out_shape=jax.ShapeDtypeStruct((R, 128), jnp.bfloat16),
    )(x, w1c)


def _l1_body(ccb, ft, f1w, f1b, w2c, x1o, y2o):
    self1 = _dot(ft[...], f1w[...]) + f1b[...]
    x1 = jnp.maximum(
        jnp.concatenate([ccb[:, :64], self1], axis=1), 0.0)
    x1o[...] = x1
    y2o[...] = _dot(x1, w2c[...]).astype(jnp.bfloat16)


def _l2_body(cc2, x1, f2w, f2b, w3c, x2o, y3o):
    x2 = (jnp.maximum(cc2[:, :64], 0.0) + _dot(x1[...], f2w[...])
          + f2b[...])
    x2o[...] = x2
    y3o[...] = _dot(x2, w3c[...]).astype(jnp.bfloat16)


def _l3_body(cc3, x2, f3w, f3b, w4c, x3o, y4o):
    x3 = _dot(x2[...], f3w[...]) + f3b[...] + cc3[:, :64]
    x3o[...] = x3
    y4o[...] = _dot(x3, w4c[...]).astype(jnp.bfloat16)


def _l4_body(cc4, x3, f4w, f4b, xo):
    xo[...] = _dot(x3[...], f4w[...]) + f4b[...] + cc4[:, :16]


def _row_spec(c):
    return pl.BlockSpec((BR, c), lambda i: (i, 0))


def _full_spec(shape):
    n = len(shape)
    return pl.BlockSpec(shape, lambda i: (0,) * n)


def _tab_spec(c):
    return pl.BlockSpec((4, BR, c), lambda i: (0, i, 0))


PERM32 = [(i // 2) + 16 * (i % 2) for i in range(32)]


def _interleave_cols(w):
    C = w.shape[-1]
    perm = [b * 32 + PERM32[i] for b in range(C // 32) for i in range(32)]
    return w[:, perm]


def _tc_fused(body, ins, outs):
    return pl.pallas_call(
        body,
        grid=(NP // BR,),
        in_specs=[s for _, s in ins],
        out_specs=[s for _, s, _ in outs],
        out_shape=[jax.ShapeDtypeStruct(sh, dt) for sh, _, dt in outs],
    )(*[a for a, _ in ins])


def kernel(dy_positions, dy_feats, box_positions, box_feats, dy_indxs,
           box_indxs, W_cc1, W_cc2, W_cc3, W_cc4,
           fc1_w, fc1_b, fc2_w, fc2_b, fc3_w, fc3_b, fc4_w, fc4_b):
    dyp = jnp.pad(dy_positions, ((0, NP - N), (0, 0)))
    dyf = jnp.pad(dy_feats, ((0, NP - N), (0, 6)))
    bxf = jnp.pad(box_feats, ((0, MPAD - M), (0, 6)))
    dyi = jnp.pad(dy_indxs, ((0, NP - N), (0, 0))).reshape(-1)
    bxi = jnp.pad(box_indxs, ((0, NP - N), (0, 0))).reshape(-1)
    dyx, dyy, dyz = dyp[:, 0], dyp[:, 1], dyp[:, 2]
    bxx, bxy, bxz = (box_positions[:, 0], box_positions[:, 1],
                     box_positions[:, 2])

    w1c = jnp.pad(_interleave_cols(
        jnp.transpose(W_cc1, (1, 0, 2)).reshape(2, 128)),
        ((0, 6), (0, 0)))
    w2c = _interleave_cols(
        jnp.transpose(W_cc2, (1, 0, 2)).reshape(96, 256))
    w3c = _interleave_cols(
        jnp.transpose(W_cc3, (1, 0, 2)).reshape(64, 256))
    w4c = _interleave_cols(
        jnp.transpose(jnp.pad(W_cc4, ((0, 0), (0, 0), (0, 29))),
                      (1, 0, 2)).reshape(64, 128))
    f1w = jnp.pad(fc1_w, ((0, 6), (0, 0)))
    f4w = jnp.pad(fc4_w, ((0, 0), (0, 13)))
    f4b = jnp.pad(fc4_b, (0, 13))

    y1d = _tc_y1(dyf, w1c)
    y1b = _tc_y1(bxf, w1c)
    dyo, dyw, ccb = _stage1(dyx, dyy, dyz, bxx, bxy, bxz, dyi, bxi,
                            y1b.reshape(MPAD * 4, 32),
                            y1d.reshape(NP * 4, 32))
    x1, y2 = _tc_fused(
        _l1_body,
        [(ccb, _row_spec(128)), (dyf, _row_spec(8)),
         (f1w, _full_spec((8, 32))),
         (fc1_b.reshape(1, 32), _full_spec((1, 32))),
         (w2c, _full_spec((96, 256)))],
        [((NP, 96), _row_spec(96), jnp.float32),
         ((NP, 256), _row_spec(256), jnp.bfloat16)])

    cc2 = _gather64(y2.reshape(NP * 4, 64), dyo, dyw)
    x2, y3 = _tc_fused(
        _l2_body,
        [(cc2, _row_spec(128)), (x1, _row_spec(96)),
         (fc2_w, _full_spec((96, 64))),
         (fc2_b.reshape(1, 64), _full_spec((1, 64))),
         (w3c, _full_spec((64, 256)))],
        [((NP, 64), _row_spec(64), jnp.float32),
         ((NP, 256), _row_spec(256), jnp.bfloat16)])

    cc3 = _gather64(y3.reshape(NP * 4, 64), dyo, dyw)
    x3, y4 = _tc_fused(
        _l3_body,
        [(cc3, _row_spec(128)), (x2, _row_spec(64)),
         (fc3_w, _full_spec((64, 64))),
         (fc3_b.reshape(1, 64), _full_spec((1, 64))),
         (w4c, _full_spec((64, 128)))],
        [((NP, 64), _row_spec(64), jnp.float32),
         ((NP, 128), _row_spec(128), jnp.bfloat16)])

    cc4 = _gather32(y4.reshape(NP * 4, 32), dyo, dyw)
    (x4,) = _tc_fused(
        _l4_body,
        [(cc4, _row_spec(128)), (x3, _row_spec(64)),
         (f4w, _full_spec((64, 16))),
         (f4b.reshape(1, 16), _full_spec((1, 16)))],
        [((NP, 16), _row_spec(16), jnp.float32)])

    return x4[:N, :3]
